# Initial kernel scaffold; baseline (speedup 1.0000x reference)
import functools

import jax
import jax.numpy as jnp
from jax import lax
from jax.experimental import pallas as pl
from jax.experimental.pallas import tpu as pltpu
from jax.experimental.pallas import tpu_sc as plsc

N = 10000
D = 128
H = 8
C = 16
NEG = 0.2

N_PAD = 10240
K = 128
TILES = 32
ROWS_PER_TILE = N_PAD // 16


def _mm_body(x_ref, wl_ref, wr_ref, attf_ref, xl_ref, xr_ref, am_ref, bm_ref):
    i = pl.program_id(0)
    x = x_ref[...]
    xl = lax.dot_general(x, wl_ref[...], (((1,), (0,)), ((), ())),
                         precision=lax.Precision.HIGHEST)
    xr = lax.dot_general(x, wr_ref[...], (((1,), (0,)), ((), ())),
                         precision=lax.Precision.HIGHEST)
    xl_ref[...] = xl
    xr_ref[...] = xr
    attf = attf_ref[...]
    ta = jnp.abs(xl) * jnp.abs(attf)
    tb = jnp.abs(xr) * jnp.abs(attf)
    col = lax.broadcasted_iota(jnp.int32, (D, H), 0)
    row = lax.broadcasted_iota(jnp.int32, (D, H), 1)
    hsel = ((col // C) == row).astype(jnp.float32)
    a_h = lax.dot_general(ta, hsel, (((1,), (0,)), ((), ())),
                          precision=lax.Precision.HIGHEST)
    b_h = lax.dot_general(tb, hsel, (((1,), (0,)), ((), ())),
                          precision=lax.Precision.HIGHEST)

    @pl.when(i == 0)
    def _():
        am_ref[...] = jnp.zeros_like(am_ref)
        bm_ref[...] = jnp.zeros_like(bm_ref)

    am_ref[...] = jnp.maximum(am_ref[...], jnp.max(a_h))
    bm_ref[...] = jnp.maximum(bm_ref[...], jnp.max(b_h))


def _run_mm(x_pad, W_l, W_r, att_flat):
    nb = N_PAD // 1024
    return pl.pallas_call(
        _mm_body,
        grid=(nb,),
        in_specs=[
            pl.BlockSpec((1024, D), lambda i: (i, 0)),
            pl.BlockSpec((D, D), lambda i: (0, 0)),
            pl.BlockSpec((D, D), lambda i: (0, 0)),
            pl.BlockSpec((1, D), lambda i: (0, 0)),
        ],
        out_specs=[
            pl.BlockSpec((1024, D), lambda i: (i, 0)),
            pl.BlockSpec((1024, D), lambda i: (i, 0)),
            pl.BlockSpec((8, D), lambda i: (0, 0)),
            pl.BlockSpec((8, D), lambda i: (0, 0)),
        ],
        out_shape=[
            jax.ShapeDtypeStruct((N_PAD, D), jnp.float32),
            jax.ShapeDtypeStruct((N_PAD, D), jnp.float32),
            jax.ShapeDtypeStruct((8, D), jnp.float32),
            jax.ShapeDtypeStruct((8, D), jnp.float32),
        ],
    )(x_pad, W_l, W_r, att_flat)


def _sc_edges(e_pad, src, dst, xl, xr, att_flat, m16):
    ept = e_pad // TILES
    mesh = plsc.VectorSubcoreMesh(core_axis_name="c", subcore_axis_name="s")

    @functools.partial(
        pl.kernel,
        mesh=mesh,
        out_type=[
            jax.ShapeDtypeStruct((2, N_PAD, D), jnp.float32),
            jax.ShapeDtypeStruct((2, N_PAD, C), jnp.float32),
        ],
        scratch_types=[
            pltpu.VMEM((K,), jnp.int32),
            pltpu.VMEM((K,), jnp.int32),
            pltpu.VMEM((K, D), jnp.float32),
            pltpu.VMEM((K, D), jnp.float32),
            pltpu.VMEM((K, D), jnp.float32),
            pltpu.VMEM((K, C), jnp.float32),
            pltpu.VMEM((D,), jnp.float32),
            pltpu.VMEM((16,), jnp.float32),
            pltpu.VMEM_SHARED((N_PAD, D), jnp.float32),
            pltpu.VMEM_SHARED((N_PAD, C), jnp.float32),
            pltpu.SemaphoreType.DMA,
            pltpu.SemaphoreType.DMA,
        ],
    )
    def k(src_hbm, dst_hbm, xl_hbm, xr_hbm, att_hbm, m_hbm,
          acc_out, den_out,
          src_v, dst_v, xl_v, xr_v, msg_v, num_v, att_v, m_v,
          acc_sp, den_sp, sem1, sem2):
        c = lax.axis_index("c")
        s = lax.axis_index("s")
        wid = s * 2 + c

        zero16 = jnp.zeros((16,), jnp.float32)

        @pl.loop(0, K)
        def _(r):
            for j in range(D // 16):
                msg_v[r, pl.ds(j * 16, 16)] = zero16
            num_v[r, :] = zero16

        for t in range(ROWS_PER_TILE // K):
            rows = pl.ds(s * ROWS_PER_TILE + t * K, K)
            pltpu.sync_copy(msg_v, acc_sp.at[rows])
            pltpu.sync_copy(num_v, den_sp.at[rows])

        pltpu.sync_copy(att_hbm, att_v)
        pltpu.sync_copy(m_hbm, m_v)
        mvec = m_v[...]
        att_h = [att_v[pl.ds(h * 16, 16)] for h in range(H)]
        lane = lax.iota(jnp.int32, 16)

        plsc.subcore_barrier()

        base_e = wid * ept

        @pl.loop(0, ept, step=K)
        def _(off0):
            off = base_e + off0
            pltpu.sync_copy(src_hbm.at[pl.ds(off, K)], src_v)
            pltpu.sync_copy(dst_hbm.at[pl.ds(off, K)], dst_v)
            cp1 = pltpu.async_copy(xl_hbm.at[src_v], xl_v, sem1)
            cp2 = pltpu.async_copy(xr_hbm.at[dst_v], xr_v, sem2)
            cp1.wait()
            cp2.wait()

            @pl.loop(0, K)
            def _(e):
                score = zero16
                a_regs = []
                for h in range(H):
                    a = xl_v[e, pl.ds(h * 16, 16)]
                    b = xr_v[e, pl.ds(h * 16, 16)]
                    u = a + b
                    lr = jnp.where(u > 0.0, u, u * NEG)
                    sh = jnp.sum(att_h[h] * lr)
                    score = jnp.where(lane == h, sh, score)
                    a_regs.append(a)
                num = jnp.exp(score - mvec)
                num_v[e, :] = num
                for h in range(H):
                    nh = num_v[e, h]
                    msg_v[e, pl.ds(h * 16, 16)] = a_regs[h] * nh

            pltpu.sync_copy(msg_v, acc_sp.at[dst_v], add=True)
            pltpu.sync_copy(num_v, den_sp.at[dst_v], add=True)

        plsc.subcore_barrier()

        for t in range(ROWS_PER_TILE // K):
            rows = pl.ds(s * ROWS_PER_TILE + t * K, K)
            pltpu.sync_copy(acc_sp.at[rows], acc_out.at[c, rows])
            pltpu.sync_copy(den_sp.at[rows], den_out.at[c, rows])

    return k(src, dst, xl, xr, att_flat, m16)


def _fin_body(acc_ref, den_ref, bias_ref, w1t_ref, w2t_ref, g_ref, b_ref,
              out_ref, cs_ref):
    p = pl.program_id(0)
    i = pl.program_id(1)
    acc = acc_ref[0] + acc_ref[1]
    den8 = den_ref[0, :, :H] + den_ref[1, :, :H]
    rden = 1.0 / den8
    col = lax.broadcasted_iota(jnp.int32, (H, D), 0)
    row = lax.broadcasted_iota(jnp.int32, (H, D), 1)
    hexp = (col == (row // C)).astype(jnp.float32)
    rdex = lax.dot_general(rden, hexp, (((1,), (0,)), ((), ())),
                           precision=lax.Precision.HIGHEST)
    out_pre = acc * rdex + bias_ref[...]

    @pl.when((p == 0) & (i == 0))
    def _():
        cs_ref[...] = jnp.zeros_like(cs_ref)

    @pl.when(p == 0)
    def _():
        cs_ref[...] = cs_ref[...] + jnp.sum(out_pre, axis=0, keepdims=True)
        out_ref[...] = out_pre

    @pl.when(p == 1)
    def _():
        y = cs_ref[...] * (1.0 / N)
        h1 = lax.dot_general(y, w1t_ref[...], (((1,), (0,)), ((), ())),
                             precision=lax.Precision.HIGHEST)
        h1 = jnp.maximum(h1, 0.0)
        g = lax.dot_general(h1, w2t_ref[...], (((1,), (0,)), ((), ())),
                            precision=lax.Precision.HIGHEST)
        gate = 1.0 / (1.0 + jnp.exp(-g))
        o = out_pre * gate
        mu = jnp.mean(o, axis=1, keepdims=True)
        var = jnp.mean((o - mu) ** 2, axis=1, keepdims=True)
        o = (o - mu) * lax.rsqrt(var + 1e-5) * g_ref[...] + b_ref[...]
        out_ref[...] = jnp.where(o > 0.0, o, o * NEG)


def _run_fin(accp, denp, bias, se_w1t, se_w2t, gamma, beta):
    nb = 10
    blk = N // nb
    return pl.pallas_call(
        _fin_body,
        grid=(2, nb),
        in_specs=[
            pl.BlockSpec((2, blk, D), lambda p, i: (0, i, 0)),
            pl.BlockSpec((2, blk, C), lambda p, i: (0, i, 0)),
            pl.BlockSpec((1, D), lambda p, i: (0, 0)),
            pl.BlockSpec((D, C), lambda p, i: (0, 0)),
            pl.BlockSpec((C, D), lambda p, i: (0, 0)),
            pl.BlockSpec((1, D), lambda p, i: (0, 0)),
            pl.BlockSpec((1, D), lambda p, i: (0, 0)),
        ],
        out_specs=pl.BlockSpec((blk, D), lambda p, i: (i, 0)),
        out_shape=jax.ShapeDtypeStruct((N, D), jnp.float32),
        scratch_shapes=[pltpu.VMEM((1, D), jnp.float32)],
    )(accp, denp, bias, se_w1t, se_w2t, gamma, beta)


def kernel(x, edge_index, W_l, W_r, att, bias, se_w1, se_w2, ln_gamma,
           ln_beta):
    n = x.shape[0]
    e = edge_index.shape[1]
    loop_idx = jnp.arange(n, dtype=jnp.int32)
    src = jnp.concatenate([edge_index[0].astype(jnp.int32), loop_idx])
    dst = jnp.concatenate([edge_index[1].astype(jnp.int32), loop_idx])
    e1 = e + n
    ept = -(-e1 // (TILES * K)) * K
    e_pad = ept * TILES
    src = jnp.pad(src, (0, e_pad - e1), constant_values=n)
    dst = jnp.pad(dst, (0, e_pad - e1), constant_values=n)

    x_pad = jnp.pad(x, ((0, N_PAD - n), (0, 0)))
    att_flat = att.reshape(1, D)

    xl, xr, amax, bmax = _run_mm(x_pad, W_l, W_r, att_flat)
    m16 = amax[0, :16] + bmax[0, :16]

    accp, denp = _sc_edges(e_pad, src, dst, xl, xr, att_flat.reshape(D), m16)

    out = _run_fin(accp, denp, bias.reshape(1, D), se_w1.T, se_w2.T,
                   ln_gamma.reshape(1, D), ln_beta.reshape(1, D))
    return out


# speedup vs baseline: 45.2426x; 45.2426x over previous
import dataclasses
import functools

import jax
import jax.numpy as jnp
from jax import lax
from jax.experimental import pallas as pl
from jax.experimental.pallas import tpu as pltpu
from jax.experimental.pallas import tpu_sc as plsc

N = 10000
D = 128
H = 8
C = 16
NEG = 0.2

N_PAD = 10240
K = 64
TILES = 32
ROWS_PER_TILE = N_PAD // 16


def _mm_body(x_ref, wl_ref, wr_ref, attf_ref, xl_ref, xr_ref, am_ref, bm_ref):
    i = pl.program_id(0)
    x = x_ref[...]
    xl = lax.dot_general(x, wl_ref[...], (((1,), (0,)), ((), ())),
                         precision=lax.Precision.HIGHEST)
    xr = lax.dot_general(x, wr_ref[...], (((1,), (0,)), ((), ())),
                         precision=lax.Precision.HIGHEST)
    xl_ref[...] = xl
    xr_ref[...] = xr
    attf = attf_ref[...]
    ta = jnp.abs(xl) * jnp.abs(attf)
    tb = jnp.abs(xr) * jnp.abs(attf)
    col = lax.broadcasted_iota(jnp.int32, (D, H), 0)
    row = lax.broadcasted_iota(jnp.int32, (D, H), 1)
    hsel = ((col // C) == row).astype(jnp.float32)
    a_h = lax.dot_general(ta, hsel, (((1,), (0,)), ((), ())),
                          precision=lax.Precision.HIGHEST)
    b_h = lax.dot_general(tb, hsel, (((1,), (0,)), ((), ())),
                          precision=lax.Precision.HIGHEST)

    @pl.when(i == 0)
    def _():
        am_ref[...] = jnp.zeros_like(am_ref)
        bm_ref[...] = jnp.zeros_like(bm_ref)

    am_ref[...] = jnp.maximum(am_ref[...], jnp.max(a_h))
    bm_ref[...] = jnp.maximum(bm_ref[...], jnp.max(b_h))


def _run_mm(x_pad, W_l, W_r, att_flat):
    nb = N_PAD // 1024
    return pl.pallas_call(
        _mm_body,
        grid=(nb,),
        in_specs=[
            pl.BlockSpec((1024, D), lambda i: (i, 0)),
            pl.BlockSpec((D, D), lambda i: (0, 0)),
            pl.BlockSpec((D, D), lambda i: (0, 0)),
            pl.BlockSpec((1, D), lambda i: (0, 0)),
        ],
        out_specs=[
            pl.BlockSpec((1024, D), lambda i: (i, 0)),
            pl.BlockSpec((1024, D), lambda i: (i, 0)),
            pl.BlockSpec((8, D), lambda i: (0, 0)),
            pl.BlockSpec((8, D), lambda i: (0, 0)),
        ],
        out_shape=[
            jax.ShapeDtypeStruct((N_PAD, D), jnp.float32),
            jax.ShapeDtypeStruct((N_PAD, D), jnp.float32),
            jax.ShapeDtypeStruct((8, D), jnp.float32),
            jax.ShapeDtypeStruct((8, D), jnp.float32),
        ],
    )(x_pad, W_l, W_r, att_flat)


NQ = N_PAD // 8


def _sc_edges(e_pad, src, dst, xl, xr, att_flat, m16, zeros128):
    ept = e_pad // TILES
    mesh = plsc.VectorSubcoreMesh(core_axis_name="c", subcore_axis_name="s")
    cp = pltpu.CompilerParams()
    if "needs_layout_passes" in pltpu.CompilerParams.__dataclass_fields__:
        cp = dataclasses.replace(cp, needs_layout_passes=False)

    @functools.partial(
        pl.kernel,
        mesh=mesh,
        compiler_params=cp,
        out_type=[
            jax.ShapeDtypeStruct((2, N_PAD, D), jnp.float32),
            jax.ShapeDtypeStruct((2, NQ, D), jnp.float32),
        ],
        scratch_types=[
            pltpu.VMEM((K,), jnp.int32),
            pltpu.VMEM((K,), jnp.int32),
            pltpu.VMEM((K,), jnp.int32),
            pltpu.VMEM((K, D), jnp.float32),
            pltpu.VMEM((K, D), jnp.float32),
            pltpu.VMEM((K, D), jnp.float32),
            pltpu.VMEM((K, D), jnp.float32),
            pltpu.VMEM((D,), jnp.float32),
            pltpu.VMEM((16,), jnp.float32),
            pltpu.VMEM_SHARED((N_PAD, D), jnp.float32),
            pltpu.VMEM_SHARED((NQ, D), jnp.float32),
            pltpu.SemaphoreType.DMA,
            pltpu.SemaphoreType.DMA,
        ],
    )
    def k(src_hbm, dst_hbm, xl_hbm, xr_hbm, att_hbm, m_hbm, z_hbm,
          acc_out, den_out,
          src_v, dst_v, dstq_v, xl_v, xr_v, msg_v, num_v, att_v, m_v,
          acc_sp, den_sp, sem1, sem2):
        c = lax.axis_index("c")
        s = lax.axis_index("s")
        wid = s * 2 + c

        zero16 = jnp.zeros((16,), jnp.float32)

        for t in range(ROWS_PER_TILE // K):
            rows = pl.ds(s * ROWS_PER_TILE + t * K, K)
            pltpu.sync_copy(z_hbm.at[rows], acc_sp.at[rows])
        qrows = pl.ds(s * (NQ // 16), NQ // 16)
        pltpu.sync_copy(z_hbm.at[qrows], den_sp.at[qrows])

        pltpu.sync_copy(att_hbm, att_v)
        pltpu.sync_copy(m_hbm, m_v)
        mvec = m_v[...]
        att_h = [att_v[pl.ds(h * 16, 16)] for h in range(H)]
        lane = lax.iota(jnp.int32, 16)

        plsc.subcore_barrier()

        base_e = wid * ept

        @pl.loop(0, ept, step=K)
        def _(off0):
            off = base_e + off0
            pltpu.sync_copy(src_hbm.at[pl.ds(off, K)], src_v)
            pltpu.sync_copy(dst_hbm.at[pl.ds(off, K)], dst_v)
            cp1 = pltpu.async_copy(xl_hbm.at[src_v], xl_v, sem1)
            cp2 = pltpu.async_copy(xr_hbm.at[dst_v], xr_v, sem2)
            for g in range(K // 16):
                sl = pl.ds(g * 16, 16)
                dstq_v[sl] = lax.shift_right_logical(dst_v[sl], 3)
            cp1.wait()
            cp2.wait()

            @pl.loop(0, K, step=16)
            def _(e0):
                dgrp = dst_v[pl.ds(e0, 16)]
                for j in range(16):
                    e = e0 + j
                    score = zero16
                    a_regs = []
                    for h in range(H):
                        a = xl_v[e, pl.ds(h * 16, 16)]
                        b = xr_v[e, pl.ds(h * 16, 16)]
                        u = a + b
                        lr = jnp.where(u > 0.0, u, u * NEG)
                        sh = jnp.sum(att_h[h] * lr)
                        score = jnp.where(lane == h, sh, score)
                        a_regs.append(a)
                    num = jnp.exp(score - mvec)
                    for g in range(H):
                        num_v[e, pl.ds(g * 16, 16)] = zero16
                    dm8 = pl.multiple_of((dgrp[j] & 7) * 16, 16)
                    num_v[e, pl.ds(dm8, 16)] = num
                    for h in range(H):
                        msg_v[e, pl.ds(h * 16, 16)] = a_regs[h] * num[h]

            pltpu.sync_copy(msg_v, acc_sp.at[dst_v], add=True)
            pltpu.sync_copy(num_v, den_sp.at[dstq_v], add=True)

        plsc.subcore_barrier()

        for t in range(ROWS_PER_TILE // K):
            rows = pl.ds(s * ROWS_PER_TILE + t * K, K)
            pltpu.sync_copy(acc_sp.at[rows], acc_out.at[c, rows])
        pltpu.sync_copy(den_sp.at[qrows], den_out.at[c, qrows])

    return k(src, dst, xl, xr, att_flat, m16, zeros128)


def _fin_body(acc_ref, den_ref, bias_ref, w1t_ref, w2t_ref, g_ref, b_ref,
              out_ref, cs_ref):
    p = pl.program_id(0)
    i = pl.program_id(1)
    acc = acc_ref[0] + acc_ref[1]
    den8 = den_ref[0, :, :H] + den_ref[1, :, :H]
    rden = 1.0 / den8
    col = lax.broadcasted_iota(jnp.int32, (H, D), 0)
    row = lax.broadcasted_iota(jnp.int32, (H, D), 1)
    hexp = (col == (row // C)).astype(jnp.float32)
    rdex = lax.dot_general(rden, hexp, (((1,), (0,)), ((), ())),
                           precision=lax.Precision.HIGHEST)
    out_pre = acc * rdex + bias_ref[...]

    @pl.when((p == 0) & (i == 0))
    def _():
        cs_ref[...] = jnp.zeros_like(cs_ref)

    @pl.when(p == 0)
    def _():
        cs_ref[...] = cs_ref[...] + jnp.sum(out_pre, axis=0, keepdims=True)
        out_ref[...] = out_pre

    @pl.when(p == 1)
    def _():
        y = cs_ref[...] * (1.0 / N)
        h1 = lax.dot_general(y, w1t_ref[...], (((1,), (0,)), ((), ())),
                             precision=lax.Precision.HIGHEST)
        h1 = jnp.maximum(h1, 0.0)
        g = lax.dot_general(h1, w2t_ref[...], (((1,), (0,)), ((), ())),
                            precision=lax.Precision.HIGHEST)
        gate = 1.0 / (1.0 + jnp.exp(-g))
        o = out_pre * gate
        mu = jnp.mean(o, axis=1, keepdims=True)
        var = jnp.mean((o - mu) ** 2, axis=1, keepdims=True)
        o = (o - mu) * lax.rsqrt(var + 1e-5) * g_ref[...] + b_ref[...]
        out_ref[...] = jnp.where(o > 0.0, o, o * NEG)


def _run_fin(accp, denp, bias, se_w1t, se_w2t, gamma, beta):
    nb = 10
    blk = N // nb
    return pl.pallas_call(
        _fin_body,
        grid=(2, nb),
        in_specs=[
            pl.BlockSpec((2, blk, D), lambda p, i: (0, i, 0)),
            pl.BlockSpec((2, blk, C), lambda p, i: (0, i, 0)),
            pl.BlockSpec((1, D), lambda p, i: (0, 0)),
            pl.BlockSpec((D, C), lambda p, i: (0, 0)),
            pl.BlockSpec((C, D), lambda p, i: (0, 0)),
            pl.BlockSpec((1, D), lambda p, i: (0, 0)),
            pl.BlockSpec((1, D), lambda p, i: (0, 0)),
        ],
        out_specs=pl.BlockSpec((blk, D), lambda p, i: (i, 0)),
        out_shape=jax.ShapeDtypeStruct((N, D), jnp.float32),
        scratch_shapes=[pltpu.VMEM((1, D), jnp.float32)],
    )(accp, denp, bias, se_w1t, se_w2t, gamma, beta)


def kernel(x, edge_index, W_l, W_r, att, bias, se_w1, se_w2, ln_gamma,
           ln_beta):
    n = x.shape[0]
    e = edge_index.shape[1]
    loop_idx = jnp.arange(n, dtype=jnp.int32)
    src = jnp.concatenate([edge_index[0].astype(jnp.int32), loop_idx])
    dst = jnp.concatenate([edge_index[1].astype(jnp.int32), loop_idx])
    e1 = e + n
    ept = -(-e1 // (TILES * K)) * K
    e_pad = ept * TILES
    src = jnp.pad(src, (0, e_pad - e1), constant_values=n)
    dst = jnp.pad(dst, (0, e_pad - e1), constant_values=n)

    x_pad = jnp.pad(x, ((0, N_PAD - n), (0, 0)))
    att_flat = att.reshape(1, D)

    xl, xr, amax, bmax = _run_mm(x_pad, W_l, W_r, att_flat)
    m16 = amax[0, :16] + bmax[0, :16]

    zeros128 = jnp.zeros((N_PAD, D), jnp.float32)
    accp, denq = _sc_edges(e_pad, src, dst, xl, xr, att_flat.reshape(D), m16,
                           zeros128)
    denp = denq.reshape(2, N_PAD, C)

    out = _run_fin(accp, denp, bias.reshape(1, D), se_w1.T, se_w2.T,
                   ln_gamma.reshape(1, D), ln_beta.reshape(1, D))
    return out
